# Initial kernel scaffold; baseline (speedup 1.0000x reference)
import functools

import jax
import jax.numpy as jnp
from jax import lax
from jax.experimental import pallas as pl
from jax.experimental.pallas import tpu as pltpu
from jax.experimental.pallas import tpu_sc as plsc

_N = 2048
_L = 16
_NC = 2
_NS = 16
_NW = _NC * _NS
_QPW = _N // _NW
_NCH = _QPW // _L

_STEPLIM = 5
_TOL = 1e-4


def _make_knn_kernel():
    mesh = plsc.VectorSubcoreMesh(
        core_axis_name="c", subcore_axis_name="s",
        num_cores=_NC, num_subcores=_NS)

    out_t = jax.ShapeDtypeStruct((_N,), jnp.float32)

    @functools.partial(
        pl.kernel,
        out_type=(out_t, out_t, out_t, out_t),
        mesh=mesh,
        scratch_types=(
            pltpu.VMEM((_N,), jnp.float32),
            pltpu.VMEM((_N,), jnp.float32),
            pltpu.VMEM((_N,), jnp.float32),
            pltpu.VMEM((_QPW,), jnp.float32),
            pltpu.VMEM((_QPW,), jnp.float32),
            pltpu.VMEM((_QPW,), jnp.float32),
            pltpu.VMEM((_QPW,), jnp.float32),
            pltpu.VMEM((_QPW,), jnp.float32),
            pltpu.VMEM((_QPW,), jnp.float32),
            pltpu.VMEM((_QPW,), jnp.float32),
        ),
    )
    def knn(qt_hbm, tt_hbm, d2_hbm, mx_hbm, my_hbm, mz_hbm,
            tx_v, ty_v, tz_v, qx_v, qy_v, qz_v, o_d2, o_mx, o_my, o_mz):
        wid = lax.axis_index("s") * _NC + lax.axis_index("c")
        base = wid * _QPW

        pltpu.sync_copy(tt_hbm.at[0], tx_v)
        pltpu.sync_copy(tt_hbm.at[1], ty_v)
        pltpu.sync_copy(tt_hbm.at[2], tz_v)
        pltpu.sync_copy(qt_hbm.at[0, pl.ds(base, _QPW)], qx_v)
        pltpu.sync_copy(qt_hbm.at[1, pl.ds(base, _QPW)], qy_v)
        pltpu.sync_copy(qt_hbm.at[2, pl.ds(base, _QPW)], qz_v)

        qx = [qx_v[pl.ds(c * _L, _L)] for c in range(_NCH)]
        qy = [qy_v[pl.ds(c * _L, _L)] for c in range(_NCH)]
        qz = [qz_v[pl.ds(c * _L, _L)] for c in range(_NCH)]

        big = jnp.full((_L,), jnp.inf, dtype=jnp.float32)
        zero = jnp.zeros((_L,), dtype=jnp.int32)
        carry0 = tuple([big] * _NCH + [zero] * _NCH)

        def body(j, carry):
            mins = list(carry[:_NCH])
            idxs = list(carry[_NCH:])
            tx = jnp.full((_L,), tx_v[j], dtype=jnp.float32)
            ty = jnp.full((_L,), ty_v[j], dtype=jnp.float32)
            tz = jnp.full((_L,), tz_v[j], dtype=jnp.float32)
            jv = jnp.full((_L,), j, dtype=jnp.int32)
            for c in range(_NCH):
                dx = qx[c] - tx
                dy = qy[c] - ty
                dz = qz[c] - tz
                d2 = dx * dx + dy * dy + dz * dz
                better = d2 < mins[c]
                mins[c] = jnp.where(better, d2, mins[c])
                idxs[c] = jnp.where(better, jv, idxs[c])
            return tuple(mins + idxs)

        carry = lax.fori_loop(0, _N, body, carry0)
        mins = carry[:_NCH]
        idxs = carry[_NCH:]

        for c in range(_NCH):
            sl = pl.ds(c * _L, _L)
            o_d2[sl] = mins[c]
            o_mx[sl] = plsc.load_gather(tx_v, [idxs[c]])
            o_my[sl] = plsc.load_gather(ty_v, [idxs[c]])
            o_mz[sl] = plsc.load_gather(tz_v, [idxs[c]])

        dst = pl.ds(base, _QPW)
        pltpu.sync_copy(o_d2, d2_hbm.at[dst])
        pltpu.sync_copy(o_mx, mx_hbm.at[dst])
        pltpu.sync_copy(o_my, my_hbm.at[dst])
        pltpu.sync_copy(o_mz, mz_hbm.at[dst])

    return knn


_knn = _make_knn_kernel()


def _rigid_transform(p1, p2):
    c1 = jnp.mean(p1, axis=-2, keepdims=True)
    c2 = jnp.mean(p2, axis=-2, keepdims=True)
    q1 = p1 - c1
    q2 = p2 - c2
    H = jnp.einsum('bni,bnj->bij', q1, q2)
    U, S, Vt = jnp.linalg.svd(H, full_matrices=False)
    V = jnp.swapaxes(Vt, -1, -2)
    Ut = jnp.swapaxes(U, -1, -2)
    d = jnp.sign(jnp.linalg.det(jnp.matmul(V, Ut)))
    D = jnp.stack([jnp.ones_like(d), jnp.ones_like(d), d], axis=-1)
    R = jnp.einsum('bij,bj,bjk->bik', V, D, Ut)
    t = c2[..., 0, :] - jnp.einsum('bij,bj->bi', R, c1[..., 0, :])
    B = p1.shape[0]
    T = jnp.zeros((B, 4, 4), dtype=p1.dtype)
    T = T.at[:, :3, :3].set(R).at[:, :3, 3].set(t).at[:, 3, 3].set(1.0)
    return T


def _apply_se3(T, pts):
    R = T[:, :3, :3]
    t = T[:, :3, 3]
    return jnp.einsum('bij,bnj->bni', R, pts) + t[:, None, :]


def kernel(p1, p2):
    p2t = p2[0].T

    err0 = jnp.zeros((1,), dtype=p1.dtype)
    done0 = jnp.array(False)
    it0 = jnp.array(0, dtype=jnp.int32)

    def cond_fn(carry):
        it, err, done, temppc = carry
        return jnp.logical_and(it <= _STEPLIM, jnp.logical_not(done))

    def body_fn(carry):
        it, err, done, temppc = carry
        it = it + 1
        qt = temppc[0].T
        d2, mx, my, mz = _knn(qt, p2t)
        vals = jnp.sqrt(d2)
        errnew = jnp.sum(vals).reshape(1)
        matched = jnp.stack([mx, my, mz], axis=-1)[None]
        T = _rigid_transform(temppc, matched)
        temppc = _apply_se3(T, temppc)
        converged = jnp.abs(err - errnew)[0] < _TOL
        err = jnp.where(converged, err, errnew)
        done = converged
        return (it, err, done, temppc)

    it_f, err_f, done_f, temppc = lax.while_loop(
        cond_fn, body_fn, (it0, err0, done0, p1))
    return _rigid_transform(p1, temppc)


# speedup vs baseline: 8.7918x; 8.7918x over previous
import functools

import jax
import jax.numpy as jnp
from jax import lax
from jax.experimental import pallas as pl
from jax.experimental.pallas import tpu as pltpu
from jax.experimental.pallas import tpu_sc as plsc

_N = 2048
_L = 16
_NC = 2
_NS = 16
_NW = _NC * _NS
_QPW = _N // _NW
_NCH = _QPW // _L
_PQ = 1

_STEPLIM = 5
_TOL = 1e-4


def _make_knn_kernel():
    mesh = plsc.VectorSubcoreMesh(
        core_axis_name="c", subcore_axis_name="s",
        num_cores=_NC, num_subcores=_NS)

    out_t = jax.ShapeDtypeStruct((_N,), jnp.float32)

    @functools.partial(
        pl.kernel,
        out_type=(out_t, out_t, out_t, out_t),
        mesh=mesh,
        compiler_params=pltpu.CompilerParams(needs_layout_passes=False),
        scratch_types=(
            pltpu.VMEM((_N,), jnp.float32),
            pltpu.VMEM((_N,), jnp.float32),
            pltpu.VMEM((_N,), jnp.float32),
            pltpu.VMEM((_QPW,), jnp.float32),
            pltpu.VMEM((_QPW,), jnp.float32),
            pltpu.VMEM((_QPW,), jnp.float32),
            pltpu.VMEM((_QPW,), jnp.float32),
            pltpu.VMEM((_QPW,), jnp.float32),
            pltpu.VMEM((_QPW,), jnp.float32),
            pltpu.VMEM((_QPW,), jnp.float32),
        ),
    )
    def knn(qx_hbm, qy_hbm, qz_hbm, tx_hbm, ty_hbm, tz_hbm,
            d2_hbm, mx_hbm, my_hbm, mz_hbm,
            tx_v, ty_v, tz_v, qx_v, qy_v, qz_v, o_d2, o_mx, o_my, o_mz):
        wid = lax.axis_index("s") * _NC + lax.axis_index("c")
        base = wid * _QPW

        pltpu.sync_copy(tx_hbm, tx_v)
        pltpu.sync_copy(ty_hbm, ty_v)
        pltpu.sync_copy(tz_hbm, tz_v)
        pltpu.sync_copy(qx_hbm.at[pl.ds(base, _QPW)], qx_v)
        pltpu.sync_copy(qy_hbm.at[pl.ds(base, _QPW)], qy_v)
        pltpu.sync_copy(qz_hbm.at[pl.ds(base, _QPW)], qz_v)

        big = jnp.full((_L,), jnp.inf, dtype=jnp.float32)
        zero = jnp.zeros((_L,), dtype=jnp.int32)

        mins = [None] * _NCH
        idxs = [None] * _NCH
        for p in range(_NCH // _PQ):
            cs = range(p * _PQ, (p + 1) * _PQ)
            qx = [qx_v[pl.ds(c * _L, _L)] for c in cs]
            qy = [qy_v[pl.ds(c * _L, _L)] for c in cs]
            qz = [qz_v[pl.ds(c * _L, _L)] for c in cs]
            carry0 = tuple([big] * _PQ + [zero] * _PQ)

            def body(ch, carry, qx=qx, qy=qy, qz=qz):
                mn = list(carry[:_PQ])
                ix = list(carry[_PQ:])
                tbase = ch * _L
                txc = tx_v[pl.ds(tbase, _L)]
                tyc = ty_v[pl.ds(tbase, _L)]
                tzc = tz_v[pl.ds(tbase, _L)]
                for l in range(_L):
                    tx = jnp.full((_L,), txc[l], dtype=jnp.float32)
                    ty = jnp.full((_L,), tyc[l], dtype=jnp.float32)
                    tz = jnp.full((_L,), tzc[l], dtype=jnp.float32)
                    jv = jnp.full((_L,), tbase + l, dtype=jnp.int32)
                    for q in range(_PQ):
                        dx = qx[q] - tx
                        dy = qy[q] - ty
                        dz = qz[q] - tz
                        d2 = dx * dx + dy * dy + dz * dz
                        better = d2 < mn[q]
                        mn[q] = jnp.where(better, d2, mn[q])
                        ix[q] = jnp.where(better, jv, ix[q])
                return tuple(mn + ix)

            carry = lax.fori_loop(0, _N // _L, body, carry0)
            for i, c in enumerate(cs):
                mins[c] = carry[i]
                idxs[c] = carry[_PQ + i]

        for c in range(_NCH):
            sl = pl.ds(c * _L, _L)
            o_d2[sl] = mins[c]
            o_mx[sl] = plsc.load_gather(tx_v, [idxs[c]])
            o_my[sl] = plsc.load_gather(ty_v, [idxs[c]])
            o_mz[sl] = plsc.load_gather(tz_v, [idxs[c]])

        dst = pl.ds(base, _QPW)
        pltpu.sync_copy(o_d2, d2_hbm.at[dst])
        pltpu.sync_copy(o_mx, mx_hbm.at[dst])
        pltpu.sync_copy(o_my, my_hbm.at[dst])
        pltpu.sync_copy(o_mz, mz_hbm.at[dst])

    return knn


_knn = _make_knn_kernel()


def _rigid_transform(p1, p2):
    c1 = jnp.mean(p1, axis=-2, keepdims=True)
    c2 = jnp.mean(p2, axis=-2, keepdims=True)
    q1 = p1 - c1
    q2 = p2 - c2
    H = jnp.einsum('bni,bnj->bij', q1, q2)
    U, S, Vt = jnp.linalg.svd(H, full_matrices=False)
    V = jnp.swapaxes(Vt, -1, -2)
    Ut = jnp.swapaxes(U, -1, -2)
    d = jnp.sign(jnp.linalg.det(jnp.matmul(V, Ut)))
    D = jnp.stack([jnp.ones_like(d), jnp.ones_like(d), d], axis=-1)
    R = jnp.einsum('bij,bj,bjk->bik', V, D, Ut)
    t = c2[..., 0, :] - jnp.einsum('bij,bj->bi', R, c1[..., 0, :])
    B = p1.shape[0]
    T = jnp.zeros((B, 4, 4), dtype=p1.dtype)
    T = T.at[:, :3, :3].set(R).at[:, :3, 3].set(t).at[:, 3, 3].set(1.0)
    return T


def _apply_se3(T, pts):
    R = T[:, :3, :3]
    t = T[:, :3, 3]
    return jnp.einsum('bij,bnj->bni', R, pts) + t[:, None, :]


def kernel(p1, p2):
    t_x, t_y, t_z = p2[0, :, 0], p2[0, :, 1], p2[0, :, 2]

    err0 = jnp.zeros((1,), dtype=p1.dtype)
    done0 = jnp.array(False)
    it0 = jnp.array(0, dtype=jnp.int32)

    def cond_fn(carry):
        it, err, done, temppc = carry
        return jnp.logical_and(it <= _STEPLIM, jnp.logical_not(done))

    def body_fn(carry):
        it, err, done, temppc = carry
        it = it + 1
        d2, mx, my, mz = _knn(temppc[0, :, 0], temppc[0, :, 1],
                              temppc[0, :, 2], t_x, t_y, t_z)
        vals = jnp.sqrt(d2)
        errnew = jnp.sum(vals).reshape(1)
        matched = jnp.stack([mx, my, mz], axis=-1)[None]
        T = _rigid_transform(temppc, matched)
        temppc = _apply_se3(T, temppc)
        converged = jnp.abs(err - errnew)[0] < _TOL
        err = jnp.where(converged, err, errnew)
        done = converged
        return (it, err, done, temppc)

    it_f, err_f, done_f, temppc = lax.while_loop(
        cond_fn, body_fn, (it0, err0, done0, p1))
    return _rigid_transform(p1, temppc)
